# baseline (speedup 1.0000x reference)
import jax
import jax.numpy as jnp
from jax import lax
from jax.experimental import pallas as pl
from jax.experimental.pallas import tpu as pltpu
from jax.experimental.pallas import tpu_sc as plsc

_NC = 2
_NS = 16
_NW = _NC * _NS
_BI = 1024
_BB = 2048


def _sc_gather(indices, table, num_indices, dim):
    span = num_indices // _NW
    nblocks = span // _BI
    npairs = nblocks // 2

    mesh = plsc.VectorSubcoreMesh(core_axis_name="c", subcore_axis_name="s")

    @pl.kernel(
        out_type=jax.ShapeDtypeStruct((num_indices, dim), table.dtype),
        mesh=mesh,
        compiler_params=pltpu.CompilerParams(use_tc_tiling_on_sc=False),
        scratch_types=[
            pltpu.VMEM((_BI,), jnp.int32),
            pltpu.VMEM((_BI,), jnp.int32),
            pltpu.VMEM((_BI, dim), table.dtype),
            pltpu.VMEM((_BI, dim), table.dtype),
            pltpu.SemaphoreType.DMA,
            pltpu.SemaphoreType.DMA,
            pltpu.SemaphoreType.DMA,
            pltpu.SemaphoreType.DMA,
            pltpu.SemaphoreType.DMA,
            pltpu.SemaphoreType.DMA,
        ],
    )
    def gather_kernel(idx_hbm, table_hbm, out_hbm,
                      idx_v0, idx_v1, rows_v0, rows_v1,
                      sem_i0, sem_i1, sem_g0, sem_g1, sem_s0, sem_s1):
        wid = lax.axis_index("s") * _NC + lax.axis_index("c")
        base = wid * span
        idx_v = (idx_v0, idx_v1)
        rows_v = (rows_v0, rows_v1)
        sem_i = (sem_i0, sem_i1)
        sem_g = (sem_g0, sem_g1)
        sem_s = (sem_s0, sem_s1)

        def issue_idx(g, b):
            off = base + jnp.minimum(g, nblocks - 1) * _BI
            pltpu.async_copy(idx_hbm.at[pl.ds(off, _BI)], idx_v[b], sem_i[b])

        def wait_idx(b):
            pltpu.make_async_copy(
                idx_hbm.at[pl.ds(base, _BI)], idx_v[b], sem_i[b]
            ).wait()

        def issue_gather(b):
            pltpu.async_copy(table_hbm.at[idx_v[b]], rows_v[b], sem_g[b])

        def drain_gather(b):
            pltpu.make_async_copy(
                table_hbm.at[pl.ds(0, _BI)], rows_v[b], sem_g[b]
            ).wait()

        def issue_store(g, b):
            off = base + g * _BI
            pltpu.async_copy(rows_v[b], out_hbm.at[pl.ds(off, _BI)], sem_s[b])

        def wait_store(b):
            pltpu.make_async_copy(
                rows_v[b], out_hbm.at[pl.ds(base, _BI)], sem_s[b]
            ).wait()

        issue_idx(0, 0)
        issue_idx(1, 1)
        wait_idx(0)
        issue_gather(0)
        wait_idx(1)
        issue_gather(1)
        drain_gather(0)
        issue_store(0, 0)
        issue_idx(2, 0)
        drain_gather(1)
        issue_store(1, 1)
        issue_idx(3, 1)

        @pl.loop(1, npairs)
        def _(i):
            g0 = 2 * i
            wait_idx(0)
            wait_store(0)
            issue_gather(0)
            wait_idx(1)
            wait_store(1)
            issue_gather(1)
            drain_gather(0)
            issue_store(g0, 0)
            issue_idx(g0 + 2, 0)
            drain_gather(1)
            issue_store(g0 + 1, 1)
            issue_idx(g0 + 3, 1)

        wait_idx(0)
        wait_idx(1)
        wait_store(0)
        wait_store(1)

    return gather_kernel(indices, table)


def _tc_transpose(rows3, hist, batch, dim):
    def body(in_ref, out_ref):
        out_ref[...] = jnp.swapaxes(in_ref[...], 1, 2)

    return pl.pallas_call(
        body,
        grid=(hist, batch // _BB),
        in_specs=[pl.BlockSpec((1, _BB, dim), lambda t, j: (t, j, 0))],
        out_specs=pl.BlockSpec((1, dim, _BB), lambda t, j: (t, 0, j)),
        out_shape=jax.ShapeDtypeStruct((hist, dim, batch), rows3.dtype),
    )(rows3)


def kernel(x, embedding):
    batch, hist = x.shape
    dim = embedding.shape[1]
    num_indices = batch * hist
    assert num_indices % (_NW * 2 * _BI) == 0 and batch % _BB == 0

    idx_flat = x.T.reshape((num_indices,)).astype(jnp.int32)
    rows = _sc_gather(idx_flat, embedding, num_indices, dim)
    rows3 = rows.reshape((hist, batch, dim))
    res = _tc_transpose(rows3, hist, batch, dim)
    return res.transpose(2, 0, 1)


# speedup vs baseline: 1.0930x; 1.0930x over previous
import jax
import jax.numpy as jnp
from jax import lax
from jax.experimental import pallas as pl
from jax.experimental.pallas import tpu as pltpu
from jax.experimental.pallas import tpu_sc as plsc

_NC = 2
_NS = 16
_NW = _NC * _NS
_BI = 1024
_TT = 512


def _sc_gather(indices, table, num_indices, dim):
    span = num_indices // _NW
    nblocks = span // _BI
    npairs = nblocks // 2

    mesh = plsc.VectorSubcoreMesh(core_axis_name="c", subcore_axis_name="s")

    @pl.kernel(
        out_type=jax.ShapeDtypeStruct((num_indices, dim), table.dtype),
        mesh=mesh,
        compiler_params=pltpu.CompilerParams(use_tc_tiling_on_sc=False),
        scratch_types=[
            pltpu.VMEM((_BI,), jnp.int32),
            pltpu.VMEM((_BI,), jnp.int32),
            pltpu.VMEM((_BI, dim), table.dtype),
            pltpu.VMEM((_BI, dim), table.dtype),
            pltpu.SemaphoreType.DMA,
            pltpu.SemaphoreType.DMA,
            pltpu.SemaphoreType.DMA,
            pltpu.SemaphoreType.DMA,
            pltpu.SemaphoreType.DMA,
            pltpu.SemaphoreType.DMA,
        ],
    )
    def gather_kernel(idx_hbm, table_hbm, out_hbm,
                      idx_v0, idx_v1, rows_v0, rows_v1,
                      sem_i0, sem_i1, sem_g0, sem_g1, sem_s0, sem_s1):
        wid = lax.axis_index("s") * _NC + lax.axis_index("c")
        base = wid * span
        idx_v = (idx_v0, idx_v1)
        rows_v = (rows_v0, rows_v1)
        sem_i = (sem_i0, sem_i1)
        sem_g = (sem_g0, sem_g1)
        sem_s = (sem_s0, sem_s1)

        def issue_idx(g, b):
            off = base + jnp.minimum(g, nblocks - 1) * _BI
            pltpu.async_copy(idx_hbm.at[pl.ds(off, _BI)], idx_v[b], sem_i[b])

        def wait_idx(b):
            pltpu.make_async_copy(
                idx_hbm.at[pl.ds(base, _BI)], idx_v[b], sem_i[b]
            ).wait()

        def issue_gather(b):
            pltpu.async_copy(table_hbm.at[idx_v[b]], rows_v[b], sem_g[b])

        def drain_gather(b):
            pltpu.make_async_copy(
                table_hbm.at[pl.ds(0, _BI)], rows_v[b], sem_g[b]
            ).wait()

        def issue_store(g, b):
            off = base + g * _BI
            pltpu.async_copy(rows_v[b], out_hbm.at[pl.ds(off, _BI)], sem_s[b])

        def wait_store(b):
            pltpu.make_async_copy(
                rows_v[b], out_hbm.at[pl.ds(base, _BI)], sem_s[b]
            ).wait()

        issue_idx(0, 0)
        issue_idx(1, 1)
        wait_idx(0)
        issue_gather(0)
        wait_idx(1)
        issue_gather(1)
        drain_gather(0)
        issue_store(0, 0)
        issue_idx(2, 0)
        drain_gather(1)
        issue_store(1, 1)
        issue_idx(3, 1)

        @pl.loop(1, npairs)
        def _(i):
            g0 = 2 * i
            wait_idx(0)
            wait_store(0)
            issue_gather(0)
            wait_idx(1)
            wait_store(1)
            issue_gather(1)
            drain_gather(0)
            issue_store(g0, 0)
            issue_idx(g0 + 2, 0)
            drain_gather(1)
            issue_store(g0 + 1, 1)
            issue_idx(g0 + 3, 1)

        wait_idx(0)
        wait_idx(1)
        wait_store(0)
        wait_store(1)

    return gather_kernel(indices, table)


def _tc_transpose(rows128, hist, batch, dim):
    pack = 128 // dim
    rq = batch // pack
    nch = rq // _TT

    def body(in_ref, out_ref, w_ref):
        i = pl.program_id(1)

        @pl.when(i == 0)
        def _():
            for j in range(nch):
                w_ref[:, _TT * j:_TT * (j + 1)] = (
                    in_ref[0, pl.ds(_TT * j, _TT), :].T
                )

        out_ref[...] = w_ref[pl.ds(i * dim, dim), :].reshape(
            1, dim, 1, rq // 128, 128
        )

    return pl.pallas_call(
        body,
        grid=(hist, pack),
        in_specs=[pl.BlockSpec((1, rq, 128), lambda t, i: (t, 0, 0))],
        out_specs=pl.BlockSpec(
            (1, dim, 1, rq // 128, 128), lambda t, i: (t, 0, i, 0, 0)
        ),
        out_shape=jax.ShapeDtypeStruct(
            (hist, dim, pack, rq // 128, 128), rows128.dtype
        ),
        scratch_shapes=[pltpu.VMEM((128, rq), rows128.dtype)],
        compiler_params=pltpu.CompilerParams(
            dimension_semantics=("parallel", "arbitrary")
        ),
    )(rows128)


def kernel(x, embedding):
    batch, hist = x.shape
    dim = embedding.shape[1]
    num_indices = batch * hist
    assert num_indices % (_NW * 2 * _BI) == 0

    pack = 128 // dim
    idx_perm = (
        x.T.reshape((hist, pack, batch // pack))
        .transpose(0, 2, 1)
        .reshape((num_indices,))
        .astype(jnp.int32)
    )
    rows = _sc_gather(idx_perm, embedding, num_indices, dim)
    rows128 = rows.reshape((hist, batch * dim // 128, 128))
    res = _tc_transpose(rows128, hist, batch, dim)
    return res.reshape((hist, dim, batch)).transpose(2, 0, 1)


# speedup vs baseline: 1.6561x; 1.5153x over previous
import jax
import jax.numpy as jnp
from jax import lax
from jax.experimental import pallas as pl
from jax.experimental.pallas import tpu as pltpu
from jax.experimental.pallas import tpu_sc as plsc

_NC = 2
_NS = 16
_NW = _NC * _NS
_BI = 1024
_TT = 512


def _sc_gather(indices, table, num_indices, dim, hist):
    span = num_indices // _NW
    nblocks = span // _BI
    npairs = nblocks // 2
    pk = 128 // dim
    nb = num_indices // hist
    nq = nb // pk
    qb = _BI // pk

    mesh = plsc.VectorSubcoreMesh(core_axis_name="c", subcore_axis_name="s")

    @pl.kernel(
        out_type=jax.ShapeDtypeStruct((num_indices, dim), table.dtype),
        mesh=mesh,
        compiler_params=pltpu.CompilerParams(
            use_tc_tiling_on_sc=False, needs_layout_passes=False
        ),
        scratch_types=[
            pltpu.VMEM((_BI,), jnp.int32),
            pltpu.VMEM((_BI,), jnp.int32),
            pltpu.VMEM((_BI,), jnp.int32),
            pltpu.VMEM((_BI,), jnp.int32),
            pltpu.VMEM((_BI, dim), table.dtype),
            pltpu.VMEM((_BI, dim), table.dtype),
            pltpu.SemaphoreType.DMA,
            pltpu.SemaphoreType.DMA,
            pltpu.SemaphoreType.DMA,
            pltpu.SemaphoreType.DMA,
            pltpu.SemaphoreType.DMA,
            pltpu.SemaphoreType.DMA,
        ],
    )
    def gather_kernel(idx_hbm, table_hbm, out_hbm,
                      idx_r0, idx_r1, idx_v0, idx_v1, rows_v0, rows_v1,
                      sem_i0, sem_i1, sem_g0, sem_g1, sem_s0, sem_s1):
        wid = lax.axis_index("s") * _NC + lax.axis_index("c")
        base = wid * span
        idx_r = (idx_r0, idx_r1)
        idx_v = (idx_v0, idx_v1)
        rows_v = (rows_v0, rows_v1)
        sem_i = (sem_i0, sem_i1)
        sem_g = (sem_g0, sem_g1)
        sem_s = (sem_s0, sem_s1)

        def issue_idx(g, b):
            j0 = base + jnp.minimum(g, nblocks - 1) * _BI
            t = j0 // nb
            rbase = pl.multiple_of(t * nb + (j0 % nb) // pk, qb)
            for i in range(pk):
                pltpu.async_copy(
                    idx_hbm.at[pl.ds(rbase + i * nq, qb)],
                    idx_r[b].at[pl.ds(i * qb, qb)],
                    sem_i[b],
                )

        def wait_idx(b):
            pltpu.make_async_copy(
                idx_hbm.at[pl.ds(base, _BI)], idx_r[b], sem_i[b]
            ).wait()

        shift = pk.bit_length() - 1

        def permute_idx(b):
            lanes = lax.iota(jnp.int32, 16)

            @pl.loop(0, _BI // 16)
            def _(k):
                j = lanes + k * 16
                src = (j & (pk - 1)) * qb + (j >> shift)
                idx_v[b][pl.ds(k * 16, 16)] = plsc.load_gather(
                    idx_r[b], [src]
                )

        def issue_gather(b):
            pltpu.async_copy(table_hbm.at[idx_v[b]], rows_v[b], sem_g[b])

        def drain_gather(b):
            pltpu.make_async_copy(
                table_hbm.at[pl.ds(0, _BI)], rows_v[b], sem_g[b]
            ).wait()

        def issue_store(g, b):
            off = base + g * _BI
            pltpu.async_copy(rows_v[b], out_hbm.at[pl.ds(off, _BI)], sem_s[b])

        def wait_store(b):
            pltpu.make_async_copy(
                rows_v[b], out_hbm.at[pl.ds(base, _BI)], sem_s[b]
            ).wait()

        issue_idx(0, 0)
        issue_idx(1, 1)
        wait_idx(0)
        permute_idx(0)
        issue_gather(0)
        wait_idx(1)
        permute_idx(1)
        issue_gather(1)
        drain_gather(0)
        issue_store(0, 0)
        issue_idx(2, 0)
        drain_gather(1)
        issue_store(1, 1)
        issue_idx(3, 1)

        @pl.loop(1, npairs)
        def _(i):
            g0 = 2 * i
            wait_idx(0)
            permute_idx(0)
            wait_store(0)
            issue_gather(0)
            wait_idx(1)
            permute_idx(1)
            wait_store(1)
            issue_gather(1)
            drain_gather(0)
            issue_store(g0, 0)
            issue_idx(g0 + 2, 0)
            drain_gather(1)
            issue_store(g0 + 1, 1)
            issue_idx(g0 + 3, 1)

        wait_idx(0)
        wait_idx(1)
        wait_store(0)
        wait_store(1)

    return gather_kernel(indices, table)


def _tc_transpose(rows128, hist, batch, dim):
    pack = 128 // dim
    rq = batch // pack
    nch = rq // _TT

    def body(in_ref, out_ref, w_ref):
        i = pl.program_id(1)

        @pl.when(i == 0)
        def _():
            for j in range(nch):
                w_ref[:, _TT * j:_TT * (j + 1)] = (
                    in_ref[0, pl.ds(_TT * j, _TT), :].T
                )

        out_ref[...] = w_ref[pl.ds(i * dim, dim), :].reshape(
            1, dim, 1, rq // 128, 128
        )

    return pl.pallas_call(
        body,
        grid=(hist, pack),
        in_specs=[pl.BlockSpec((1, rq, 128), lambda t, i: (t, 0, 0))],
        out_specs=pl.BlockSpec(
            (1, dim, 1, rq // 128, 128), lambda t, i: (t, 0, i, 0, 0)
        ),
        out_shape=jax.ShapeDtypeStruct(
            (hist, dim, pack, rq // 128, 128), rows128.dtype
        ),
        scratch_shapes=[pltpu.VMEM((128, rq), rows128.dtype)],
        compiler_params=pltpu.CompilerParams(
            dimension_semantics=("parallel", "arbitrary")
        ),
    )(rows128)


def kernel(x, embedding):
    batch, hist = x.shape
    dim = embedding.shape[1]
    num_indices = batch * hist
    assert num_indices % (_NW * 2 * _BI) == 0

    idx_flat = x.T.reshape((num_indices,)).astype(jnp.int32)
    rows = _sc_gather(idx_flat, embedding, num_indices, dim, hist)
    rows128 = rows.reshape((hist, batch * dim // 128, 128))
    res = _tc_transpose(rows128, hist, batch, dim)
    return res.reshape((hist, dim, batch)).transpose(2, 0, 1)


# speedup vs baseline: 2.1067x; 1.2721x over previous
import jax
import jax.numpy as jnp
from jax import lax
from jax.experimental import pallas as pl
from jax.experimental.pallas import tpu as pltpu
from jax.experimental.pallas import tpu_sc as plsc

_NC = 2
_NS = 16
_NW = _NC * _NS
_BI = 1024
_TT = 512


def _sc_gather(indices, table, num_indices, dim, hist):
    span = num_indices // _NW
    nblocks = span // _BI
    npairs = nblocks // 2
    pk = 128 // dim
    nb = num_indices // hist
    nq = nb // pk
    qb = _BI // pk

    mesh = plsc.VectorSubcoreMesh(core_axis_name="c", subcore_axis_name="s")

    @pl.kernel(
        out_type=jax.ShapeDtypeStruct((num_indices, dim), table.dtype),
        mesh=mesh,
        compiler_params=pltpu.CompilerParams(
            use_tc_tiling_on_sc=False, needs_layout_passes=False
        ),
        scratch_types=[
            pltpu.VMEM((_BI,), jnp.int32),
            pltpu.VMEM((_BI,), jnp.int32),
            pltpu.VMEM((_BI,), jnp.int32),
            pltpu.VMEM((_BI,), jnp.int32),
            pltpu.VMEM((_BI, dim), table.dtype),
            pltpu.VMEM((_BI, dim), table.dtype),
            pltpu.SemaphoreType.DMA,
            pltpu.SemaphoreType.DMA,
            pltpu.SemaphoreType.DMA,
            pltpu.SemaphoreType.DMA,
            pltpu.SemaphoreType.DMA,
            pltpu.SemaphoreType.DMA,
        ],
    )
    def gather_kernel(idx_hbm, table_hbm, out_hbm,
                      idx_r0, idx_r1, idx_v0, idx_v1, rows_v0, rows_v1,
                      sem_i0, sem_i1, sem_g0, sem_g1, sem_s0, sem_s1):
        wid = lax.axis_index("s") * _NC + lax.axis_index("c")
        base = wid * span
        idx_r = (idx_r0, idx_r1)
        idx_v = (idx_v0, idx_v1)
        rows_v = (rows_v0, rows_v1)
        sem_i = (sem_i0, sem_i1)
        sem_g = (sem_g0, sem_g1)
        sem_s = (sem_s0, sem_s1)

        def issue_idx(g, b):
            j0 = base + jnp.minimum(g, nblocks - 1) * _BI
            t = j0 // nb
            rbase = pl.multiple_of(t * nb + (j0 % nb) // pk, qb)
            for i in range(pk):
                pltpu.async_copy(
                    idx_hbm.at[pl.ds(rbase + i * nq, qb)],
                    idx_r[b].at[pl.ds(i * qb, qb)],
                    sem_i[b],
                )

        def wait_idx(b):
            pltpu.make_async_copy(
                idx_hbm.at[pl.ds(base, _BI)], idx_r[b], sem_i[b]
            ).wait()

        shift = pk.bit_length() - 1

        def permute_idx(b):
            lanes = lax.iota(jnp.int32, 16)

            @pl.loop(0, _BI // 16)
            def _(k):
                j = lanes + k * 16
                src = (j & (pk - 1)) * qb + (j >> shift)
                idx_v[b][pl.ds(k * 16, 16)] = plsc.load_gather(
                    idx_r[b], [src]
                )

        def issue_gather(b):
            pltpu.async_copy(table_hbm.at[idx_v[b]], rows_v[b], sem_g[b])

        def drain_gather(b):
            pltpu.make_async_copy(
                table_hbm.at[pl.ds(0, _BI)], rows_v[b], sem_g[b]
            ).wait()

        def issue_store(g, b):
            off = base + g * _BI
            pltpu.async_copy(rows_v[b], out_hbm.at[pl.ds(off, _BI)], sem_s[b])

        def wait_store(b):
            pltpu.make_async_copy(
                rows_v[b], out_hbm.at[pl.ds(base, _BI)], sem_s[b]
            ).wait()

        issue_idx(0, 0)
        issue_idx(1, 1)
        wait_idx(0)
        permute_idx(0)
        issue_gather(0)
        wait_idx(1)
        permute_idx(1)
        issue_gather(1)
        drain_gather(0)
        issue_store(0, 0)
        issue_idx(2, 0)
        drain_gather(1)
        issue_store(1, 1)
        issue_idx(3, 1)

        @pl.loop(1, npairs)
        def _(i):
            g0 = 2 * i
            wait_idx(0)
            permute_idx(0)
            wait_store(0)
            issue_gather(0)
            wait_idx(1)
            permute_idx(1)
            wait_store(1)
            issue_gather(1)
            drain_gather(0)
            issue_store(g0, 0)
            issue_idx(g0 + 2, 0)
            drain_gather(1)
            issue_store(g0 + 1, 1)
            issue_idx(g0 + 3, 1)

        wait_idx(0)
        wait_idx(1)
        wait_store(0)
        wait_store(1)

    return gather_kernel(indices, table)


def _tc_transpose(rows128, hist, batch, dim):
    pack = 128 // dim
    rq = batch // pack
    nch = rq // _TT
    ntc = 2
    span_t = hist // ntc
    dt = rows128.dtype

    mesh = pltpu.create_tensorcore_mesh("core", num_cores=ntc)

    @pl.kernel(
        out_type=jax.ShapeDtypeStruct(
            (hist, dim, pack, rq // 128, 128), dt
        ),
        mesh=mesh,
        scratch_types=[
            pltpu.VMEM((rq, 128), dt),
            pltpu.VMEM((rq, 128), dt),
            pltpu.VMEM((128, rq // 128, 128), dt),
            pltpu.VMEM((128, rq // 128, 128), dt),
            pltpu.SemaphoreType.DMA,
            pltpu.SemaphoreType.DMA,
            pltpu.SemaphoreType.DMA,
            pltpu.SemaphoreType.DMA,
        ],
    )
    def tc_kernel(rows_hbm, out_hbm, in0, in1, w0, w1,
                  sem_i0, sem_i1, sem_o0, sem_o1):
        t0 = lax.axis_index("core") * span_t
        in_v = (in0, in1)
        w_v = (w0, w1)
        sem_i = (sem_i0, sem_i1)
        sem_o = (sem_o0, sem_o1)

        def issue_in(k, b):
            t = t0 + jnp.minimum(k, span_t - 1)
            pltpu.async_copy(rows_hbm.at[t], in_v[b], sem_i[b])

        def wait_in(b):
            pltpu.make_async_copy(rows_hbm.at[0], in_v[b], sem_i[b]).wait()

        def compute(b):
            for j in range(nch):
                w_v[b][:, pack * j:pack * (j + 1), :] = (
                    in_v[b][pl.ds(_TT * j, _TT), :].T.reshape(
                        128, pack, 128
                    )
                )

        def issue_outs(t, b):
            for i in range(pack):
                pltpu.async_copy(
                    w_v[b].at[pl.ds(dim * i, dim)],
                    out_hbm.at[t, :, i],
                    sem_o[b],
                )

        def wait_outs(b):
            for _ in range(pack):
                pltpu.make_async_copy(
                    w_v[b].at[pl.ds(0, dim)],
                    out_hbm.at[0, :, 0],
                    sem_o[b],
                ).wait()

        issue_in(0, 0)
        issue_in(1, 1)
        wait_in(0)
        compute(0)
        issue_outs(t0, 0)
        issue_in(2, 0)
        wait_in(1)
        compute(1)
        issue_outs(t0 + 1, 1)
        issue_in(3, 1)

        @pl.loop(1, span_t // 2)
        def _(k2):
            for b in range(2):
                k = 2 * k2 + b
                wait_in(b)
                wait_outs(b)
                compute(b)
                issue_outs(t0 + k, b)
                issue_in(k + 2, b)

        wait_in(0)
        wait_in(1)
        wait_outs(0)
        wait_outs(1)

    return tc_kernel(rows128)


def kernel(x, embedding):
    batch, hist = x.shape
    dim = embedding.shape[1]
    num_indices = batch * hist
    assert num_indices % (_NW * 2 * _BI) == 0

    idx_flat = x.T.reshape((num_indices,)).astype(jnp.int32)
    rows = _sc_gather(idx_flat, embedding, num_indices, dim, hist)
    rows128 = rows.reshape((hist, batch * dim // 128, 128))
    res = _tc_transpose(rows128, hist, batch, dim)
    return res.reshape((hist, dim, batch)).transpose(2, 0, 1)
